# Initial kernel scaffold; baseline (speedup 1.0000x reference)
import jax
import jax.numpy as jnp
from jax.experimental import pallas as pl


def kernel(x, table, W, b):
    raise NotImplementedError("write your pallas kernel here")


# speedup vs baseline: 2.2817x; 2.2817x over previous
import functools

import jax
import jax.numpy as jnp
from jax import lax
from jax.experimental import pallas as pl
from jax.experimental.pallas import tpu as pltpu
from jax.experimental.pallas import tpu_sc as plsc

LANES = 16


def _sc_pool(x, table):
    B, SEQ = x.shape
    _, D = table.shape
    NC, NS = 2, 16
    NW = NC * NS
    RPW = B // NW
    C0 = 128
    C1 = SEQ - C0

    mesh = plsc.VectorSubcoreMesh(core_axis_name="c", subcore_axis_name="s")

    @functools.partial(
        pl.kernel,
        out_type=jax.ShapeDtypeStruct((B, D), jnp.float32),
        mesh=mesh,
        scratch_types=[
            pltpu.VMEM((RPW, SEQ), jnp.int32),
            pltpu.VMEM((SEQ, D), jnp.float32),
            pltpu.VMEM((SEQ, D), jnp.float32),
            pltpu.VMEM((RPW, D), jnp.float32),
            pltpu.SemaphoreType.DMA,
            pltpu.SemaphoreType.DMA,
        ],
        compiler_params=pltpu.CompilerParams(
            use_tc_tiling_on_sc=False, needs_layout_passes=False),
    )
    def k(x_hbm, tab_hbm, mean_hbm, xv, rows0, rows1, meanv, sem0, sem1):
        wid = lax.axis_index("s") * NC + lax.axis_index("c")
        base = wid * RPW
        pltpu.sync_copy(x_hbm.at[pl.ds(base, RPW)], xv)

        def issue(i, rows, sem):
            pltpu.async_copy(
                tab_hbm.at[xv.at[i, pl.ds(0, C0)]], rows.at[pl.ds(0, C0)], sem)
            pltpu.async_copy(
                tab_hbm.at[xv.at[i, pl.ds(C0, C1)]], rows.at[pl.ds(C0, C1)], sem)

        def drain(rows, sem):
            pltpu.make_async_copy(tab_hbm.at[pl.ds(0, SEQ)], rows, sem).wait()

        def compute(i, rows):
            def body(j, carry):
                a0, a1 = carry
                a0 = a0 + rows[j, pl.ds(0, LANES)]
                a1 = a1 + rows[j, pl.ds(LANES, LANES)]
                return a0, a1

            z = jnp.zeros((LANES,), jnp.float32)
            a0, a1 = lax.fori_loop(0, SEQ, body, (z, z), unroll=8)

            cnt = jnp.zeros((LANES,), jnp.int32)
            for kk in range(SEQ // LANES):
                chunk = xv[i, pl.ds(kk * LANES, LANES)]
                cnt = cnt + plsc.all_reduce_population_count(chunk != 0)
            rem = SEQ - (SEQ // LANES) * LANES
            if rem:
                lane = lax.iota(jnp.int32, LANES)
                last = xv[i, pl.ds(SEQ - LANES, LANES)]
                cnt = cnt + plsc.all_reduce_population_count(
                    (lane >= LANES - rem) & (last != 0))
            denom = jnp.maximum(cnt.astype(jnp.float32),
                                jnp.ones((LANES,), jnp.float32))
            meanv[i, pl.ds(0, LANES)] = a0 / denom
            meanv[i, pl.ds(LANES, LANES)] = a1 / denom

        issue(0, rows0, sem0)

        def body2(t, carry):
            i0 = t * 2
            issue(i0 + 1, rows1, sem1)
            drain(rows0, sem0)
            compute(i0, rows0)

            @pl.when(i0 + 2 < RPW)
            def _():
                issue(i0 + 2, rows0, sem0)

            drain(rows1, sem1)
            compute(i0 + 1, rows1)
            return carry

        lax.fori_loop(0, RPW // 2, body2, 0)
        pltpu.sync_copy(meanv, mean_hbm.at[pl.ds(base, RPW)])

    return k(x, table)


def _mm(mean, W, b2):
    B, D = mean.shape
    C, _ = W.shape
    BLK = 256

    def mmk(m_ref, w_ref, b_ref, o_ref):
        o_ref[...] = lax.dot_general(
            m_ref[...], w_ref[...],
            dimension_numbers=(((1,), (1,)), ((), ())),
            preferred_element_type=jnp.float32,
        ) + b_ref[...]

    return pl.pallas_call(
        mmk,
        grid=(B // BLK,),
        in_specs=[
            pl.BlockSpec((BLK, D), lambda i: (i, 0)),
            pl.BlockSpec((C, D), lambda i: (0, 0)),
            pl.BlockSpec((1, C), lambda i: (0, 0)),
        ],
        out_specs=pl.BlockSpec((BLK, C), lambda i: (i, 0)),
        out_shape=jax.ShapeDtypeStruct((B, C), jnp.float32),
    )(mean, W, b2)


@jax.jit
def kernel(x, table, W, b):
    x = x.astype(jnp.int32)
    mean = _sc_pool(x, table)
    return _mm(mean, W, b.reshape(1, -1))


# speedup vs baseline: 2.6982x; 1.1825x over previous
import functools

import jax
import jax.numpy as jnp
from jax import lax
from jax.experimental import pallas as pl
from jax.experimental.pallas import tpu as pltpu
from jax.experimental.pallas import tpu_sc as plsc

LANES = 16


def _sc_pool(x, table):
    B, SEQ = x.shape
    _, D = table.shape
    NC, NS = 2, 16
    NW = NC * NS
    RPW = B // NW
    C0 = 128
    C1 = SEQ - C0

    mesh = plsc.VectorSubcoreMesh(core_axis_name="c", subcore_axis_name="s")

    @functools.partial(
        pl.kernel,
        out_type=jax.ShapeDtypeStruct((B, D), jnp.float32),
        mesh=mesh,
        scratch_types=[
            pltpu.VMEM((RPW, SEQ), jnp.int32),
            pltpu.VMEM((SEQ, D), jnp.float32),
            pltpu.VMEM((SEQ, D), jnp.float32),
            pltpu.VMEM((RPW, D), jnp.float32),
            pltpu.SemaphoreType.DMA,
            pltpu.SemaphoreType.DMA,
        ],
        compiler_params=pltpu.CompilerParams(
            use_tc_tiling_on_sc=False, needs_layout_passes=False),
    )
    def k(x_hbm, tab_hbm, mean_hbm, xv, rows0, rows1, meanv, sem0, sem1):
        wid = lax.axis_index("s") * NC + lax.axis_index("c")
        base = wid * RPW
        pltpu.sync_copy(x_hbm.at[pl.ds(base, RPW)], xv)

        def issue(i, rows, sem):
            pltpu.async_copy(
                tab_hbm.at[xv.at[i, pl.ds(0, C0)]], rows.at[pl.ds(0, C0)], sem)
            pltpu.async_copy(
                tab_hbm.at[xv.at[i, pl.ds(C0, C1)]], rows.at[pl.ds(C0, C1)], sem)

        def drain(rows, sem):
            pltpu.make_async_copy(tab_hbm.at[pl.ds(0, SEQ)], rows, sem).wait()

        def compute(i, rows):
            def body(j, carry):
                a0, a1 = carry
                a0 = a0 + rows[j, pl.ds(0, LANES)]
                a1 = a1 + rows[j, pl.ds(LANES, LANES)]
                return a0, a1

            z = jnp.zeros((LANES,), jnp.float32)
            a0, a1 = lax.fori_loop(0, SEQ, body, (z, z), unroll=8)

            cnt = jnp.zeros((LANES,), jnp.int32)
            for kk in range(SEQ // LANES):
                chunk = xv[i, pl.ds(kk * LANES, LANES)]
                cnt = cnt + plsc.all_reduce_population_count(chunk != 0)
            rem = SEQ - (SEQ // LANES) * LANES
            if rem:
                lane = lax.iota(jnp.int32, LANES)
                last = xv[i, pl.ds(SEQ - LANES, LANES)]
                cnt = cnt + plsc.all_reduce_population_count(
                    (lane >= LANES - rem) & (last != 0))
            denom = jnp.maximum(cnt.astype(jnp.float32),
                                jnp.ones((LANES,), jnp.float32))
            meanv[i, pl.ds(0, LANES)] = a0 / denom
            meanv[i, pl.ds(LANES, LANES)] = a1 / denom

        issue(0, rows0, sem0)

        def body2(t, carry):
            i0 = t * 2
            issue(i0 + 1, rows1, sem1)
            drain(rows0, sem0)
            compute(i0, rows0)

            @pl.when(i0 + 2 < RPW)
            def _():
                issue(i0 + 2, rows0, sem0)

            drain(rows1, sem1)
            compute(i0 + 1, rows1)
            return carry

        lax.fori_loop(0, RPW // 2, body2, 0)
        pltpu.sync_copy(meanv, mean_hbm.at[pl.ds(base, RPW)])

    return k(x, table)


def _tc_relayout(tabT):
    _, V = tabT.shape
    BLKC = 7936
    G = -(-V // BLKC)

    def body(in_ref, o_ref):
        piece = in_ref[...].T
        t3 = piece.reshape(BLKC // 4, 4, 32)
        o_ref[...] = jnp.concatenate(
            [t3[:, 0, :], t3[:, 1, :], t3[:, 2, :], t3[:, 3, :]], axis=-1)

    return pl.pallas_call(
        body,
        grid=(G,),
        in_specs=[pl.BlockSpec((32, BLKC), lambda i: (0, i))],
        out_specs=pl.BlockSpec((BLKC // 4, 128), lambda i: (i, 0)),
        out_shape=jax.ShapeDtypeStruct((V // 4, 128), jnp.float32),
    )(tabT)


def _mm(mean, W, b2):
    B, D = mean.shape
    C, _ = W.shape
    BLK = 256

    def mmk(m_ref, w_ref, b_ref, o_ref):
        o_ref[...] = lax.dot_general(
            m_ref[...], w_ref[...],
            dimension_numbers=(((1,), (1,)), ((), ())),
            preferred_element_type=jnp.float32,
        ) + b_ref[...]

    return pl.pallas_call(
        mmk,
        grid=(B // BLK,),
        in_specs=[
            pl.BlockSpec((BLK, D), lambda i: (i, 0)),
            pl.BlockSpec((C, D), lambda i: (0, 0)),
            pl.BlockSpec((1, C), lambda i: (0, 0)),
        ],
        out_specs=pl.BlockSpec((BLK, C), lambda i: (i, 0)),
        out_shape=jax.ShapeDtypeStruct((B, C), jnp.float32),
    )(mean, W, b2)


@jax.jit
def kernel(x, table, W, b):
    x = x.astype(jnp.int32)
    V, D = table.shape
    tab_lin = _tc_relayout(table.T).reshape(V, D)
    mean = _sc_pool(x, tab_lin)
    return _mm(mean, W, b.reshape(1, -1))
